# baseline (speedup 1.0000x reference)
import functools

import jax
import jax.numpy as jnp
from jax import lax
from jax.experimental import pallas as pl
from jax.experimental.pallas import tpu as pltpu
from jax.experimental.pallas import tpu_sc as plsc

_NC = 2
_NS = 16
_NW = _NC * _NS
_CHUNK = 128
_NBUF = 4


@functools.cache
def _make_gather(n_rows, emb, dtype):
    assert n_rows % (_NW * _CHUNK) == 0
    n_chunks = n_rows // (_NW * _CHUNK)
    per_w = n_chunks * _CHUNK
    mesh = plsc.VectorSubcoreMesh(core_axis_name="c", subcore_axis_name="s")

    @functools.partial(
        pl.kernel,
        out_type=jax.ShapeDtypeStruct((n_rows, emb), dtype),
        mesh=mesh,
        scratch_types=[
            pltpu.VMEM((n_chunks, _CHUNK), jnp.int32),
            *[pltpu.VMEM((_CHUNK, emb), dtype) for _ in range(_NBUF)],
            pltpu.SemaphoreType.DMA,
            pltpu.SemaphoreType.DMA,
        ],
        compiler_params=pltpu.CompilerParams(use_tc_tiling_on_sc=False),
    )
    def gather(table_hbm, idx_hbm, out_hbm, idx_v, *rest):
        bufs, (g_sem, w_sem) = rest[:_NBUF], rest[_NBUF:]
        wid = lax.axis_index("s") * _NC + lax.axis_index("c")
        pltpu.sync_copy(idx_hbm.at[wid], idx_v)
        base = wid * per_w

        gathers = [None] * n_chunks
        writes = [None] * n_chunks
        for c in range(n_chunks):
            b = c % _NBUF
            if writes[c - _NBUF] is not None:
                writes[c - _NBUF].wait()
                writes[c - _NBUF] = None
            gathers[c] = pltpu.async_copy(table_hbm.at[idx_v.at[c]], bufs[b], g_sem)
            d = c - (_NBUF - 1)
            if d >= 0:
                gathers[d].wait()
                writes[d] = pltpu.async_copy(
                    bufs[d % _NBUF], out_hbm.at[pl.ds(base + d * _CHUNK, _CHUNK)], w_sem
                )
        for d in range(max(0, n_chunks - (_NBUF - 1)), n_chunks):
            gathers[d].wait()
            writes[d] = pltpu.async_copy(
                bufs[d % _NBUF], out_hbm.at[pl.ds(base + d * _CHUNK, _CHUNK)], w_sem
            )
        for w in writes:
            if w is not None:
                w.wait()

    return gather


def _mlp_body(flat_ref, w1_ref, b1_ref, w2_ref, b2_ref, out_ref):
    h = jnp.tanh(
        jnp.dot(flat_ref[...], w1_ref[...], preferred_element_type=jnp.float32)
        + b1_ref[...]
    )
    out_ref[...] = (
        jnp.dot(h.astype(jnp.bfloat16), w2_ref[...], preferred_element_type=jnp.float32)
        + b2_ref[...]
    )


@functools.cache
def _make_mlp(batch, d_in, d_hidden, d_out, bm):
    grid = (batch // bm,)
    return pl.pallas_call(
        _mlp_body,
        grid=grid,
        in_specs=[
            pl.BlockSpec((bm, d_in), lambda i: (i, 0)),
            pl.BlockSpec((d_in, d_hidden), lambda i: (0, 0)),
            pl.BlockSpec((1, d_hidden), lambda i: (0, 0)),
            pl.BlockSpec((d_hidden, d_out), lambda i: (0, 0)),
            pl.BlockSpec((1, d_out), lambda i: (0, 0)),
        ],
        out_specs=pl.BlockSpec((bm, d_out), lambda i: (i, 0)),
        out_shape=jax.ShapeDtypeStruct((batch, d_out), jnp.float32),
    )


def kernel(x, table, W1, b1, W2, b2):
    batch, window = x.shape
    emb = table.shape[1]
    n_rows = batch * window
    tbl = jax.lax.bitcast_convert_type(
        table.astype(jnp.bfloat16).reshape(table.shape[0], emb // 2, 2), jnp.int32
    )
    idx = x.astype(jnp.int32).reshape(_NW, n_rows // (_NW * _CHUNK), _CHUNK)
    gathered = _make_gather(n_rows, emb // 2, jnp.int32)(tbl, idx)
    flat = jax.lax.bitcast_convert_type(gathered, jnp.bfloat16).reshape(
        batch, window * emb
    )
    mlp = _make_mlp(batch, window * emb, W1.shape[1], W2.shape[1], 2048)
    return mlp(
        flat,
        W1.astype(jnp.bfloat16),
        b1.reshape(1, -1),
        W2.astype(jnp.bfloat16),
        b2.reshape(1, -1),
    )


# speedup vs baseline: 6.1076x; 6.1076x over previous
import functools

import jax
import jax.numpy as jnp
from jax import lax
from jax.experimental import pallas as pl
from jax.experimental.pallas import tpu as pltpu
from jax.experimental.pallas import tpu_sc as plsc

_NC = 2
_NS = 16
_NW = _NC * _NS
_CHUNK = 128
_NBUF = 4


@functools.cache
def _make_gather(n_rows, emb, dtype):
    assert n_rows % (_NW * _CHUNK) == 0
    n_chunks = n_rows // (_NW * _CHUNK)
    per_w = n_chunks * _CHUNK
    mesh = plsc.VectorSubcoreMesh(core_axis_name="c", subcore_axis_name="s")

    @functools.partial(
        pl.kernel,
        out_type=jax.ShapeDtypeStruct((n_rows, emb), dtype),
        mesh=mesh,
        scratch_types=[
            pltpu.VMEM((n_chunks, _CHUNK), jnp.int32),
            *[pltpu.VMEM((_CHUNK, emb), dtype) for _ in range(_NBUF)],
            pltpu.SemaphoreType.DMA,
            pltpu.SemaphoreType.DMA,
        ],
        compiler_params=pltpu.CompilerParams(use_tc_tiling_on_sc=False),
    )
    def gather(table_hbm, idx_hbm, out_hbm, idx_v, *rest):
        bufs, (g_sem, w_sem) = rest[:_NBUF], rest[_NBUF:]
        wid = lax.axis_index("s") * _NC + lax.axis_index("c")
        pltpu.sync_copy(idx_hbm.at[wid], idx_v)
        base = wid * per_w

        gathers = [None] * n_chunks
        writes = [None] * n_chunks

        def start_write(d):
            gathers[d].wait()
            writes[d] = pltpu.async_copy(
                bufs[d % _NBUF], out_hbm.at[pl.ds(base + d * _CHUNK, _CHUNK)], w_sem
            )

        for c in range(n_chunks):
            if c - _NBUF >= 0:
                writes[c - _NBUF].wait()
            gathers[c] = pltpu.async_copy(
                table_hbm.at[idx_v.at[c]], bufs[c % _NBUF], g_sem
            )
            if c - (_NBUF - 1) >= 0:
                start_write(c - (_NBUF - 1))
        for d in range(max(0, n_chunks - (_NBUF - 1)), n_chunks):
            start_write(d)
        for d in range(n_chunks - _NBUF, n_chunks):
            if d >= 0:
                writes[d].wait()

    return gather


def _mlp_body(flat_ref, w1_ref, b1_ref, w2_ref, b2_ref, out_ref):
    flat = flat_ref[...].astype(jnp.bfloat16)
    w1 = w1_ref[...].astype(jnp.bfloat16)
    h = jnp.tanh(
        jnp.dot(flat, w1, preferred_element_type=jnp.float32) + b1_ref[...]
    )
    w2 = w2_ref[...].astype(jnp.bfloat16)
    out_ref[...] = (
        jnp.dot(h.astype(jnp.bfloat16), w2, preferred_element_type=jnp.float32)
        + b2_ref[...]
    )


@functools.cache
def _make_mlp(batch, d_in, d_hidden, d_out, bm):
    grid = (batch // bm,)
    return pl.pallas_call(
        _mlp_body,
        grid=grid,
        in_specs=[
            pl.BlockSpec((bm, d_in), lambda i: (i, 0)),
            pl.BlockSpec((d_in, d_hidden), lambda i: (0, 0)),
            pl.BlockSpec((1, d_hidden), lambda i: (0, 0)),
            pl.BlockSpec((d_hidden, d_out), lambda i: (0, 0)),
            pl.BlockSpec((1, d_out), lambda i: (0, 0)),
        ],
        out_specs=pl.BlockSpec((bm, d_out), lambda i: (i, 0)),
        out_shape=jax.ShapeDtypeStruct((batch, d_out), jnp.float32),
    )


def kernel(x, table, W1, b1, W2, b2):
    batch, window = x.shape
    emb = table.shape[1]
    n_rows = batch * window
    idx = x.astype(jnp.int32).reshape(_NW, n_rows // (_NW * _CHUNK), _CHUNK)
    gathered = _make_gather(n_rows, emb, jnp.float32)(table, idx)
    flat = gathered.reshape(batch, window * emb)
    mlp = _make_mlp(batch, window * emb, W1.shape[1], W2.shape[1], 2048)
    return mlp(flat, W1, b1.reshape(1, -1), W2, b2.reshape(1, -1))
